# baseline (speedup 1.0000x reference)
import functools

import jax
import jax.numpy as jnp
from jax import lax
from jax.experimental import pallas as pl
from jax.experimental.pallas import tpu as pltpu
from jax.experimental.pallas import tpu_sc as plsc

B = 16384
F = 26
D = 32
OUT_D = 33
NW = 32
RPW = B // NW
CB = 64
NCHUNK = RPW // CB
CR = CB * F
B_TBL = 1000000
BW = 512
NBLK = 999936 // BW
BPW = 63
TAIL0 = NBLK * BW
NTAIL = B_TBL - TAIL0

_mesh = plsc.VectorSubcoreMesh(core_axis_name="c", subcore_axis_name="s")

_GATHER_DNUMS = lax.GatherDimensionNumbers(
    offset_dims=(), collapsed_slice_dims=(0,), start_index_map=(0,)
)


def _permute(v, idx):
    return lax.gather(
        v, idx[:, None], _GATHER_DNUMS, slice_sizes=(1,),
        mode=lax.GatherScatterMode.PROMISE_IN_BOUNDS,
    )


def _t16(a, lane):
    for sh in (1, 2, 4, 8):
        b = [None] * 16
        for i in range(16):
            src = _permute(a[i ^ sh], lane ^ sh)
            if i & sh == 0:
                b[i] = jnp.where((lane & sh) == 0, a[i], src)
            else:
                b[i] = jnp.where((lane & sh) == 0, src, a[i])
        a = b
    return a


@functools.partial(
    pl.kernel,
    out_type=jax.ShapeDtypeStruct((B_TBL * D,), jnp.float32),
    mesh=_mesh,
    scratch_types=[
        pltpu.VMEM((D, BW), jnp.float32),
        pltpu.VMEM((D, BW), jnp.float32),
        pltpu.VMEM((D, BW), jnp.float32),
        pltpu.VMEM((BW * D,), jnp.float32),
        pltpu.VMEM((BW * D,), jnp.float32),
        pltpu.VMEM((BW * D,), jnp.float32),
        pltpu.SemaphoreType.DMA((3,)),
        pltpu.SemaphoreType.DMA((3,)),
    ],
    compiler_params=pltpu.CompilerParams(use_tc_tiling_on_sc=True),
)
def _relayout(tt_hbm, out_hbm, blk0, blk1, blk2, row0, row1, row2, isem,
              osem):
    wid = lax.axis_index("s") * 2 + lax.axis_index("c")
    lane = lax.iota(jnp.int32, 16)
    blks = (blk0, blk1, blk2)
    rows = (row0, row1, row2)

    def blk_start(k):
        return jnp.minimum(wid * BPW + k, NBLK - 1) * BW

    def in_copy(kb, k):
        pltpu.make_async_copy(
            tt_hbm.at[pl.ds(0, D), pl.ds(blk_start(k), BW)],
            blks[kb], isem.at[kb]).start()

    def transpose(kb, k, ngrp):
        blk = blks[kb]
        row = rows[kb]

        def grp(gi, carry):
            c0 = gi * 16
            lo = [blk[d, pl.ds(c0, 16)] for d in range(16)]
            hi = [blk[d + 16, pl.ds(c0, 16)] for d in range(16)]
            lo = _t16(lo, lane)
            hi = _t16(hi, lane)
            for j in range(16):
                o = (c0 + j) * D
                row[pl.ds(o, 16)] = lo[j]
                row[pl.ds(o + 16, 16)] = hi[j]
            return carry

        lax.fori_loop(0, ngrp, grp, 0)

    def out_copy(kb, k, n):
        pltpu.make_async_copy(
            rows[kb].at[pl.ds(0, n * D)],
            out_hbm.at[pl.ds(blk_start(k) * D, n * D)], osem.at[kb]).start()

    in_copy(0, 0)
    in_copy(1, 1)
    in_copy(2, 2)

    def step_triple(j, carry):
        for kb in (0, 1, 2):
            k = j * 3 + kb
            pltpu.make_async_copy(
                tt_hbm.at[pl.ds(0, D), pl.ds(blk_start(k), BW)],
                blks[kb], isem.at[kb]).wait()

            @pl.when(k >= 3)
            def _():
                pltpu.make_async_copy(
                    rows[kb].at[pl.ds(0, BW * D)],
                    out_hbm.at[pl.ds(blk_start(k - 3) * D, BW * D)],
                    osem.at[kb]).wait()

            transpose(kb, k, BW // 16)
            out_copy(kb, k, BW)

            @pl.when(k + 3 < BPW)
            def _():
                in_copy(kb, k + 3)

        return carry

    lax.fori_loop(0, BPW // 3, step_triple, 0)
    for tail_k in (BPW - 3, BPW - 2, BPW - 1):
        pltpu.make_async_copy(
            rows[tail_k % 3].at[pl.ds(0, BW * D)],
            out_hbm.at[pl.ds(blk_start(tail_k) * D, BW * D)],
            osem.at[tail_k % 3]).wait()

    @pl.when(wid == NW - 1)
    def _():
        for d in range(D):
            pltpu.make_async_copy(tt_hbm.at[d, pl.ds(TAIL0, NTAIL)],
                                  blk0.at[d, pl.ds(0, NTAIL)],
                                  isem.at[0]).start()
        for d in range(D):
            pltpu.make_async_copy(tt_hbm.at[d, pl.ds(TAIL0, NTAIL)],
                                  blk0.at[d, pl.ds(0, NTAIL)],
                                  isem.at[0]).wait()
        transpose(0, 0, NTAIL // 16)
        pltpu.sync_copy(row0.at[pl.ds(0, NTAIL * D)],
                        out_hbm.at[pl.ds(TAIL0 * D, NTAIL * D)])


@functools.partial(
    pl.kernel,
    out_type=jax.ShapeDtypeStruct((B * OUT_D,), jnp.float32),
    mesh=_mesh,
    scratch_types=[
        pltpu.VMEM((RPW * F,), jnp.int32),
        pltpu.VMEM((CR, D), jnp.float32),
        pltpu.VMEM((CR, D), jnp.float32),
        pltpu.VMEM((CB * OUT_D,), jnp.float32),
        pltpu.VMEM((CB * OUT_D,), jnp.float32),
        pltpu.VMEM((32,), jnp.float32),
        pltpu.SemaphoreType.DMA((2,)),
        pltpu.SemaphoreType.DMA((2,)),
    ],
    compiler_params=pltpu.CompilerParams(use_tc_tiling_on_sc=False),
)
def _tower(ids_hbm, table_hbm, w_hbm, out_hbm, idx_v, rows0, rows1, slab0,
           slab1, w_v, gsem, osem):
    wid = lax.axis_index("s") * 2 + lax.axis_index("c")
    base = wid * (RPW * F)
    pltpu.sync_copy(ids_hbm.at[pl.ds(base, RPW * F)], idx_v)
    pltpu.sync_copy(w_hbm, w_v)
    w0 = w_v[pl.ds(0, 16)]
    w1 = w_v[pl.ds(16, 16)]
    rows = (rows0, rows1)
    slabs = (slab0, slab1)

    def gcopy(kb, g):
        return pltpu.make_async_copy(
            table_hbm.at[idx_v.at[pl.ds(g * CR, CR)]], rows[kb], gsem.at[kb])

    def ocopy(kb, g):
        row0 = wid * RPW + g * CB
        return pltpu.make_async_copy(
            slabs[kb], out_hbm.at[pl.ds(row0 * OUT_D, CB * OUT_D)],
            osem.at[kb])

    def compute(kb):
        rv = rows[kb]
        sv = slabs[kb]

        def row_body(b, carry2):
            r0 = b * F
            acc0 = jnp.zeros((16,), jnp.float32)
            acc1 = jnp.zeros((16,), jnp.float32)
            q = jnp.zeros((16,), jnp.float32)
            for f in range(F):
                e0 = rv[r0 + f, pl.ds(0, 16)]
                e1 = rv[r0 + f, pl.ds(16, 16)]
                acc0 = acc0 + e0
                acc1 = acc1 + e1
                q = q + e0 * e0 + e1 * e1
            t = acc0 * w0 + acc1 * w1 + 0.5 * (acc0 * acc0 + acc1 * acc1) - 0.5 * q
            lane = lax.iota(jnp.int32, 16)
            for sh in (8, 4, 2, 1):
                t = t + _permute(t, lane ^ sh)
            o = b * OUT_D
            sv[pl.ds(o, 16)] = t
            sv[pl.ds(o + 1, 16)] = acc0
            sv[pl.ds(o + 17, 16)] = acc1
            return carry2

        lax.fori_loop(0, CB, row_body, 0)

    gcopy(0, 0).start()
    gcopy(1, 1).start()

    def chunk_pair(j, carry):
        for kb in (0, 1):
            g = j * 2 + kb
            gcopy(kb, g).wait()

            @pl.when(g >= 2)
            def _():
                ocopy(kb, g - 2).wait()

            compute(kb)

            @pl.when(g + 2 < NCHUNK)
            def _():
                gcopy(kb, g + 2).start()

            ocopy(kb, g).start()

        return carry

    lax.fori_loop(0, NCHUNK // 2, chunk_pair, 0)
    ocopy(0, NCHUNK - 2).wait()
    ocopy(1, NCHUNK - 1).wait()


def kernel(item_feature_ids, emb_table, linear_w):
    ids = item_feature_ids.astype(jnp.int32).reshape(-1)
    w_flat = linear_w.reshape(-1)
    tbl_lin = _relayout(emb_table.T).reshape(B_TBL, D)
    return _tower(ids, tbl_lin, w_flat).reshape(B, OUT_D)


# speedup vs baseline: 1.0719x; 1.0719x over previous
import functools

import jax
import jax.numpy as jnp
from jax import lax
from jax.experimental import pallas as pl
from jax.experimental.pallas import tpu as pltpu
from jax.experimental.pallas import tpu_sc as plsc

B = 16384
F = 26
D = 32
OUT_D = 33
NW = 32
RPW = B // NW
CB = 64
NCHUNK = RPW // CB
CR = CB * F
B_TBL = 1000000
BW = 256
NBLK = 999936 // BW
BPW = 123
TAIL0 = NBLK * BW
NTAIL = B_TBL - TAIL0

_mesh = plsc.VectorSubcoreMesh(core_axis_name="c", subcore_axis_name="s")

_GATHER_DNUMS = lax.GatherDimensionNumbers(
    offset_dims=(), collapsed_slice_dims=(0,), start_index_map=(0,)
)


def _permute(v, idx):
    return lax.gather(
        v, idx[:, None], _GATHER_DNUMS, slice_sizes=(1,),
        mode=lax.GatherScatterMode.PROMISE_IN_BOUNDS,
    )


def _t16(a, lane):
    for sh in (1, 2, 4, 8):
        b = [None] * 16
        for i in range(16):
            src = _permute(a[i ^ sh], lane ^ sh)
            if i & sh == 0:
                b[i] = jnp.where((lane & sh) == 0, a[i], src)
            else:
                b[i] = jnp.where((lane & sh) == 0, src, a[i])
        a = b
    return a


@functools.partial(
    pl.kernel,
    out_type=jax.ShapeDtypeStruct((B_TBL * D,), jnp.float32),
    mesh=_mesh,
    scratch_types=[
        pltpu.VMEM((D, BW), jnp.float32),
        pltpu.VMEM((D, BW), jnp.float32),
        pltpu.VMEM((D, BW), jnp.float32),
        pltpu.VMEM((BW * D,), jnp.float32),
        pltpu.VMEM((BW * D,), jnp.float32),
        pltpu.VMEM((BW * D,), jnp.float32),
        pltpu.SemaphoreType.DMA((3,)),
        pltpu.SemaphoreType.DMA((3,)),
    ],
    compiler_params=pltpu.CompilerParams(use_tc_tiling_on_sc=True),
)
def _relayout(tt_hbm, out_hbm, blk0, blk1, blk2, row0, row1, row2, isem,
              osem):
    wid = lax.axis_index("s") * 2 + lax.axis_index("c")
    lane = lax.iota(jnp.int32, 16)
    blks = (blk0, blk1, blk2)
    rows = (row0, row1, row2)

    def blk_start(k):
        return jnp.minimum(wid * BPW + k, NBLK - 1) * BW

    def in_copy(kb, k):
        pltpu.make_async_copy(
            tt_hbm.at[pl.ds(0, D), pl.ds(blk_start(k), BW)],
            blks[kb], isem.at[kb]).start()

    def transpose(kb, k, ngrp):
        blk = blks[kb]
        row = rows[kb]

        def grp(gi, carry):
            c0 = gi * 16
            lo = [blk[d, pl.ds(c0, 16)] for d in range(16)]
            hi = [blk[d + 16, pl.ds(c0, 16)] for d in range(16)]
            lo = _t16(lo, lane)
            hi = _t16(hi, lane)
            for j in range(16):
                o = (c0 + j) * D
                row[pl.ds(o, 16)] = lo[j]
                row[pl.ds(o + 16, 16)] = hi[j]
            return carry

        lax.fori_loop(0, ngrp, grp, 0)

    def out_copy(kb, k, n):
        pltpu.make_async_copy(
            rows[kb].at[pl.ds(0, n * D)],
            out_hbm.at[pl.ds(blk_start(k) * D, n * D)], osem.at[kb]).start()

    in_copy(0, 0)
    in_copy(1, 1)
    in_copy(2, 2)

    def step_triple(j, carry):
        for kb in (0, 1, 2):
            k = j * 3 + kb
            pltpu.make_async_copy(
                tt_hbm.at[pl.ds(0, D), pl.ds(blk_start(k), BW)],
                blks[kb], isem.at[kb]).wait()

            @pl.when(k >= 3)
            def _():
                pltpu.make_async_copy(
                    rows[kb].at[pl.ds(0, BW * D)],
                    out_hbm.at[pl.ds(blk_start(k - 3) * D, BW * D)],
                    osem.at[kb]).wait()

            transpose(kb, k, BW // 16)
            out_copy(kb, k, BW)

            @pl.when(k + 3 < BPW)
            def _():
                in_copy(kb, k + 3)

        return carry

    lax.fori_loop(0, BPW // 3, step_triple, 0)
    for tail_k in (BPW - 3, BPW - 2, BPW - 1):
        pltpu.make_async_copy(
            rows[tail_k % 3].at[pl.ds(0, BW * D)],
            out_hbm.at[pl.ds(blk_start(tail_k) * D, BW * D)],
            osem.at[tail_k % 3]).wait()

    @pl.when(wid == NW - 1)
    def _():
        for d in range(D):
            pltpu.make_async_copy(tt_hbm.at[d, pl.ds(TAIL0, NTAIL)],
                                  blk0.at[d, pl.ds(0, NTAIL)],
                                  isem.at[0]).start()
        for d in range(D):
            pltpu.make_async_copy(tt_hbm.at[d, pl.ds(TAIL0, NTAIL)],
                                  blk0.at[d, pl.ds(0, NTAIL)],
                                  isem.at[0]).wait()
        transpose(0, 0, NTAIL // 16)
        pltpu.sync_copy(row0.at[pl.ds(0, NTAIL * D)],
                        out_hbm.at[pl.ds(TAIL0 * D, NTAIL * D)])


@functools.partial(
    pl.kernel,
    out_type=jax.ShapeDtypeStruct((B * OUT_D,), jnp.float32),
    mesh=_mesh,
    scratch_types=[
        pltpu.VMEM((RPW * F,), jnp.int32),
        pltpu.VMEM((CR, D), jnp.float32),
        pltpu.VMEM((CR, D), jnp.float32),
        pltpu.VMEM((CB * OUT_D,), jnp.float32),
        pltpu.VMEM((CB * OUT_D,), jnp.float32),
        pltpu.VMEM((32,), jnp.float32),
        pltpu.SemaphoreType.DMA((2,)),
        pltpu.SemaphoreType.DMA((2,)),
    ],
    compiler_params=pltpu.CompilerParams(use_tc_tiling_on_sc=False),
)
def _tower(ids_hbm, table_hbm, w_hbm, out_hbm, idx_v, rows0, rows1, slab0,
           slab1, w_v, gsem, osem):
    wid = lax.axis_index("s") * 2 + lax.axis_index("c")
    base = wid * (RPW * F)
    pltpu.sync_copy(ids_hbm.at[pl.ds(base, RPW * F)], idx_v)
    pltpu.sync_copy(w_hbm, w_v)
    w0 = w_v[pl.ds(0, 16)]
    w1 = w_v[pl.ds(16, 16)]
    rows = (rows0, rows1)
    slabs = (slab0, slab1)

    def gcopy(kb, g):
        return pltpu.make_async_copy(
            table_hbm.at[idx_v.at[pl.ds(g * CR, CR)]], rows[kb], gsem.at[kb])

    def ocopy(kb, g):
        row0 = wid * RPW + g * CB
        return pltpu.make_async_copy(
            slabs[kb], out_hbm.at[pl.ds(row0 * OUT_D, CB * OUT_D)],
            osem.at[kb])

    def compute(kb):
        rv = rows[kb]
        sv = slabs[kb]

        def row_body(b, carry2):
            r0 = b * F
            acc0 = jnp.zeros((16,), jnp.float32)
            acc1 = jnp.zeros((16,), jnp.float32)
            q = jnp.zeros((16,), jnp.float32)
            for f in range(F):
                e0 = rv[r0 + f, pl.ds(0, 16)]
                e1 = rv[r0 + f, pl.ds(16, 16)]
                acc0 = acc0 + e0
                acc1 = acc1 + e1
                q = q + e0 * e0 + e1 * e1
            t = acc0 * w0 + acc1 * w1 + 0.5 * (acc0 * acc0 + acc1 * acc1) - 0.5 * q
            lane = lax.iota(jnp.int32, 16)
            for sh in (8, 4, 2, 1):
                t = t + _permute(t, lane ^ sh)
            o = b * OUT_D
            sv[pl.ds(o, 16)] = t
            sv[pl.ds(o + 1, 16)] = acc0
            sv[pl.ds(o + 17, 16)] = acc1
            return carry2

        lax.fori_loop(0, CB, row_body, 0)

    gcopy(0, 0).start()
    gcopy(1, 1).start()

    def chunk_pair(j, carry):
        for kb in (0, 1):
            g = j * 2 + kb
            gcopy(kb, g).wait()

            @pl.when(g >= 2)
            def _():
                ocopy(kb, g - 2).wait()

            compute(kb)

            @pl.when(g + 2 < NCHUNK)
            def _():
                gcopy(kb, g + 2).start()

            ocopy(kb, g).start()

        return carry

    lax.fori_loop(0, NCHUNK // 2, chunk_pair, 0)
    ocopy(0, NCHUNK - 2).wait()
    ocopy(1, NCHUNK - 1).wait()


@functools.partial(
    pl.kernel,
    out_type=jax.ShapeDtypeStruct((OUT_D, B), jnp.float32),
    mesh=_mesh,
    scratch_types=[
        pltpu.VMEM((RPW * OUT_D,), jnp.float32),
        pltpu.VMEM((OUT_D, RPW), jnp.float32),
    ],
    compiler_params=pltpu.CompilerParams(use_tc_tiling_on_sc=True),
)
def _outxpose(flat_hbm, out_hbm, in_v, slab_v):
    wid = lax.axis_index("s") * 2 + lax.axis_index("c")
    lane = lax.iota(jnp.int32, 16)
    pltpu.sync_copy(flat_hbm.at[pl.ds(wid * RPW * OUT_D, RPW * OUT_D)], in_v)

    def grp(g, carry):
        o0 = g * 16 * OUT_D
        va = [in_v[pl.ds(o0 + j * OUT_D, 16)] for j in range(16)]
        vb = [in_v[pl.ds(o0 + j * OUT_D + 16, 16)] for j in range(16)]
        vc = [in_v[pl.ds(o0 + j * OUT_D + 17, 16)] for j in range(16)]
        ta = _t16(va, lane)
        tb = _t16(vb, lane)
        tc = _t16(vc, lane)
        for c in range(16):
            slab_v[c, pl.ds(g * 16, 16)] = ta[c]
            slab_v[16 + c, pl.ds(g * 16, 16)] = tb[c]
        slab_v[32, pl.ds(g * 16, 16)] = tc[15]
        return carry

    lax.fori_loop(0, RPW // 16, grp, 0)
    pltpu.sync_copy(slab_v,
                    out_hbm.at[pl.ds(0, OUT_D), pl.ds(wid * RPW, RPW)])


def kernel(item_feature_ids, emb_table, linear_w):
    ids = item_feature_ids.astype(jnp.int32).reshape(-1)
    w_flat = linear_w.reshape(-1)
    tbl_lin = _relayout(emb_table.T).reshape(B_TBL, D)
    flat = _tower(ids, tbl_lin, w_flat)
    return _outxpose(flat).T
